# baseline (speedup 1.0000x reference)
import functools

import jax
import jax.numpy as jnp
from jax import lax
from jax.experimental import pallas as pl
from jax.experimental.pallas import tpu as pltpu
from jax.experimental.pallas import tpu_sc as plsc

EMBED = 64
NC = 2
NS = 16
NW = NC * NS

CHUNK = 128


@functools.lru_cache(maxsize=None)
def _build(n_tokens):
    b_per_w = n_tokens // NW
    n_chunks = b_per_w // CHUNK
    assert b_per_w * NW == n_tokens and n_chunks * CHUNK == b_per_w
    mesh = plsc.VectorSubcoreMesh(core_axis_name="c", subcore_axis_name="s")

    @functools.partial(
        pl.kernel,
        mesh=mesh,
        out_type=jax.ShapeDtypeStruct((n_tokens, EMBED), jnp.float32),
        scratch_types=[
            pltpu.VMEM((b_per_w,), jnp.int32),
            pltpu.VMEM((CHUNK, EMBED), jnp.float32),
            pltpu.SemaphoreType.DMA,
        ],
        compiler_params=pltpu.CompilerParams(use_tc_tiling_on_sc=False),
    )
    def emb(tok_hbm, table_hbm, out_hbm, idx_v, rows_v, sem):
        wid = lax.axis_index("s") * NC + lax.axis_index("c")
        base = wid * b_per_w
        pltpu.sync_copy(tok_hbm.at[pl.ds(base, b_per_w)], idx_v)

        def chunk_body(j, carry):
            off = j * CHUNK
            pltpu.async_copy(
                table_hbm.at[idx_v.at[pl.ds(off, CHUNK)]], rows_v, sem
            ).wait()
            pltpu.sync_copy(rows_v, out_hbm.at[pl.ds(base + off, CHUNK)])
            return carry

        lax.fori_loop(0, n_chunks, chunk_body, 0)

    return emb


def kernel(tokens, table):
    shape = tokens.shape
    flat = tokens.reshape(-1).astype(jnp.int32)
    out = _build(flat.shape[0])(flat, table)
    return out.reshape(*shape, EMBED)


# speedup vs baseline: 1.0875x; 1.0875x over previous
import functools

import jax
import jax.numpy as jnp
from jax import lax
from jax.experimental import pallas as pl
from jax.experimental.pallas import tpu as pltpu
from jax.experimental.pallas import tpu_sc as plsc

EMBED = 64
NC = 2
NS = 16
NW = NC * NS

CHUNK = 128
NBUF = 5


@functools.lru_cache(maxsize=None)
def _build(n_tokens):
    b_per_w = n_tokens // NW
    n_chunks = b_per_w // CHUNK
    n_super = n_chunks // NBUF
    assert b_per_w * NW == n_tokens
    assert n_chunks * CHUNK == b_per_w
    assert n_super * NBUF == n_chunks
    mesh = plsc.VectorSubcoreMesh(core_axis_name="c", subcore_axis_name="s")

    @functools.partial(
        pl.kernel,
        mesh=mesh,
        out_type=jax.ShapeDtypeStruct((n_tokens, EMBED), jnp.float32),
        scratch_types=[
            pltpu.VMEM((b_per_w,), jnp.int32),
            pltpu.VMEM((NBUF, CHUNK, EMBED), jnp.float32),
        ] + [pltpu.SemaphoreType.DMA] * (2 * NBUF),
        compiler_params=pltpu.CompilerParams(use_tc_tiling_on_sc=False),
    )
    def emb(tok_hbm, table_hbm, out_hbm, idx_v, rows_v, *sems):
        gsem = sems[:NBUF]
        wsem = sems[NBUF:]
        wid = lax.axis_index("s") * NC + lax.axis_index("c")
        base = wid * b_per_w
        pltpu.sync_copy(tok_hbm.at[pl.ds(base, b_per_w)], idx_v)

        def _gather_args(j, b):
            return (table_hbm.at[idx_v.at[pl.ds(j * CHUNK, CHUNK)]],
                    rows_v.at[b], gsem[b])

        def _writeback_args(j, b):
            return (rows_v.at[b],
                    out_hbm.at[pl.ds(base + j * CHUNK, CHUNK)], wsem[b])

        def gather_start(j, b):
            pltpu.async_copy(*_gather_args(j, b))

        def gather_wait(j, b):
            pltpu.make_async_copy(*_gather_args(j, b)).wait()

        def writeback_start(j, b):
            pltpu.async_copy(*_writeback_args(j, b))

        def writeback_wait(j, b):
            pltpu.make_async_copy(*_writeback_args(j, b)).wait()

        for b in range(NBUF):
            gather_start(b, b)

        def superstep(g, carry):
            for b in range(NBUF):
                j = g * NBUF + b
                gather_wait(j, b)
                writeback_start(j, b)
            for b in range(NBUF):
                j = g * NBUF + b
                writeback_wait(j, b)
                gather_start(j + NBUF, b)
            return carry

        lax.fori_loop(0, n_super - 1, superstep, 0)

        g = n_super - 1
        for b in range(NBUF):
            j = g * NBUF + b
            gather_wait(j, b)
            writeback_start(j, b)
        for b in range(NBUF):
            writeback_wait(g * NBUF + b, b)

    return emb


def kernel(tokens, table):
    shape = tokens.shape
    flat = tokens.reshape(-1).astype(jnp.int32)
    out = _build(flat.shape[0])(flat, table)
    return out.reshape(*shape, EMBED)
